# Initial kernel scaffold; baseline (speedup 1.0000x reference)
import functools
import jax
import jax.numpy as jnp
from jax import lax
from jax.experimental import pallas as pl
from jax.experimental.pallas import tpu as pltpu
from jax.experimental.pallas import tpu_sc as plsc

_N = 10000
_E = 320000
_D = 128
_C = 64

_NSC = 2
_NSUB = 16
_NW = _NSC * _NSUB

_N_PAD = 10240
_ROWS_TILE = _N_PAD // _NSUB
_CHUNK = 128
_CH_PER_TILE = -(-_E // (_NW * _CHUNK))
_E_TILE = _CH_PER_TILE * _CHUNK
_E_PAD = _NW * _E_TILE

_mesh = plsc.VectorSubcoreMesh(core_axis_name="c", subcore_axis_name="s")


def _deg_body(dst_hbm, out_hbm, dst_v, hist_v):
    c = lax.axis_index("c")
    s = lax.axis_index("s")
    w = c * _NSUB + s

    zeros16 = jnp.zeros((16,), jnp.float32)

    def zinit(i, _):
        hist_v[pl.ds(i * 16, 16)] = zeros16
        return 0

    lax.fori_loop(0, _N_PAD // 16, zinit, 0)

    pltpu.sync_copy(dst_hbm.at[w], dst_v)

    ones16 = jnp.ones((16,), jnp.float32)

    def body(i, _):
        idx = dst_v[pl.ds(i * 16, 16)]
        plsc.addupdate_scatter(hist_v, [idx], ones16)
        return 0

    lax.fori_loop(0, _E_TILE // 16, body, 0)

    pltpu.sync_copy(hist_v, out_hbm.at[c, s])


@jax.jit
def _deg_call(dst_flat):
    k = functools.partial(
        pl.kernel,
        mesh=_mesh,
        out_type=jax.ShapeDtypeStruct((_NSC, _NSUB, _N_PAD), jnp.float32),
        scratch_types=[
            pltpu.VMEM((_E_TILE,), jnp.int32),
            pltpu.VMEM((_N_PAD,), jnp.float32),
        ],
    )(_deg_body)
    return k(dst_flat)


def _hop_body(y_hbm, src_hbm, dst_hbm, zeros_hbm, out_hbm,
              src_v, dst_v, rows_v, acc_sh, sem):
    c = lax.axis_index("c")
    s = lax.axis_index("s")
    w = c * _NSUB + s

    pltpu.sync_copy(zeros_hbm.at[pl.ds(s * _ROWS_TILE, _ROWS_TILE)],
                    acc_sh.at[pl.ds(s * _ROWS_TILE, _ROWS_TILE)])
    pltpu.sync_copy(src_hbm.at[w], src_v)
    pltpu.sync_copy(dst_hbm.at[w], dst_v)
    plsc.subcore_barrier()

    def body(j, _):
        pltpu.async_copy(y_hbm.at[src_v.at[j]], rows_v, sem).wait()
        pltpu.sync_copy(rows_v, acc_sh.at[dst_v.at[j]], add=True)
        return 0

    lax.fori_loop(0, _CH_PER_TILE, body, 0)

    plsc.subcore_barrier()
    pltpu.sync_copy(acc_sh.at[pl.ds(s * _ROWS_TILE, _ROWS_TILE)],
                    out_hbm.at[c, pl.ds(s * _ROWS_TILE, _ROWS_TILE)])


@jax.jit
def _hop_call(y, srcp, dstp, zeros_pad):
    k = functools.partial(
        pl.kernel,
        mesh=_mesh,
        out_type=jax.ShapeDtypeStruct((_NSC, _N_PAD, _C), jnp.float32),
        scratch_types=[
            pltpu.VMEM((_CH_PER_TILE, _CHUNK), jnp.int32),
            pltpu.VMEM((_CH_PER_TILE, _CHUNK), jnp.int32),
            pltpu.VMEM((_CHUNK, _C), jnp.float32),
            pltpu.VMEM_SHARED((_N_PAD, _C), jnp.float32),
            pltpu.SemaphoreType.DMA,
        ],
    )(_hop_body)
    return k(y, srcp, dstp, zeros_pad)


_BLK = 256


def _prep_body(feat_ref, w_ref, degp_ref, y0_ref):
    deg = jnp.sum(degp_ref[...], axis=(0, 1))
    norm = lax.rsqrt(jnp.maximum(deg, 1.0))
    acc = jnp.dot(feat_ref[...], w_ref[...],
                  preferred_element_type=jnp.float32)
    y0_ref[...] = acc * norm[:, None]


@jax.jit
def _prep_call(featp, W, degp):
    return pl.pallas_call(
        _prep_body,
        grid=(_N_PAD // _BLK,),
        in_specs=[
            pl.BlockSpec((_BLK, _D), lambda i: (i, 0)),
            pl.BlockSpec((_D, _C), lambda i: (0, 0)),
            pl.BlockSpec((_NSC, _NSUB, _BLK), lambda i: (0, 0, i)),
        ],
        out_specs=pl.BlockSpec((_BLK, _C), lambda i: (i, 0)),
        out_shape=jax.ShapeDtypeStruct((_N_PAD, _C), jnp.float32),
    )(featp, W, degp)


def _mid_body(p_ref, degp_ref, y_ref):
    deg = jnp.maximum(jnp.sum(degp_ref[...], axis=(0, 1)), 1.0)
    y_ref[...] = (p_ref[0] + p_ref[1]) * (1.0 / deg)[:, None]


@jax.jit
def _mid_call(p, degp):
    return pl.pallas_call(
        _mid_body,
        grid=(_N_PAD // _BLK,),
        in_specs=[
            pl.BlockSpec((_NSC, _BLK, _C), lambda i: (0, i, 0)),
            pl.BlockSpec((_NSC, _NSUB, _BLK), lambda i: (0, 0, i)),
        ],
        out_specs=pl.BlockSpec((_BLK, _C), lambda i: (i, 0)),
        out_shape=jax.ShapeDtypeStruct((_N_PAD, _C), jnp.float32),
    )(p, degp)


def _fin_body(p_ref, degp_ref, out_ref, logits_ref):
    deg = jnp.maximum(jnp.sum(degp_ref[...], axis=(0, 1)), 1.0)
    norm = lax.rsqrt(deg)
    logits = (p_ref[0] + p_ref[1]) * norm[:, None]
    logits_ref[...] = logits
    m = jnp.max(logits, axis=1, keepdims=True)
    e = jnp.exp(logits - m)
    out_ref[...] = e / jnp.sum(e, axis=1, keepdims=True)


@jax.jit
def _fin_call(p, degp):
    return pl.pallas_call(
        _fin_body,
        grid=(_N_PAD // _BLK,),
        in_specs=[
            pl.BlockSpec((_NSC, _BLK, _C), lambda i: (0, i, 0)),
            pl.BlockSpec((_NSC, _NSUB, _BLK), lambda i: (0, 0, i)),
        ],
        out_specs=[
            pl.BlockSpec((_BLK, _C), lambda i: (i, 0)),
            pl.BlockSpec((_BLK, _C), lambda i: (i, 0)),
        ],
        out_shape=[
            jax.ShapeDtypeStruct((_N_PAD, _C), jnp.float32),
            jax.ShapeDtypeStruct((_N_PAD, _C), jnp.float32),
        ],
    )(p, degp)


def kernel(features, edge_index, W):
    src = edge_index[0]
    dst = edge_index[1]
    pad_idx = jnp.full((_E_PAD - _E,), _N_PAD - 1, jnp.int32)
    srcp = jnp.concatenate([src, pad_idx]).reshape(_NW, _CH_PER_TILE, _CHUNK)
    dstp = jnp.concatenate([dst, pad_idx]).reshape(_NW, _CH_PER_TILE, _CHUNK)
    dst_flat = dstp.reshape(_NW, _E_TILE)
    featp = jnp.pad(features, ((0, _N_PAD - _N), (0, 0)))
    zeros_pad = jnp.zeros((_N_PAD, _C), jnp.float32)

    degp = _deg_call(dst_flat)
    y0 = _prep_call(featp, W, degp)
    p1 = _hop_call(y0, srcp, dstp, zeros_pad)
    y1 = _mid_call(p1, degp)
    p2 = _hop_call(y1, srcp, dstp, zeros_pad)
    out_pad, logits_pad = _fin_call(p2, degp)
    return out_pad[:_N], logits_pad[:_N]


# speedup vs baseline: 6.2017x; 6.2017x over previous
import functools
import jax
import jax.numpy as jnp
from jax import lax
from jax.experimental import pallas as pl
from jax.experimental.pallas import tpu as pltpu
from jax.experimental.pallas import tpu_sc as plsc

_N = 10000
_E = 320000
_D = 128
_C = 64

_NSC = 2
_NSUB = 16
_NW = _NSC * _NSUB

_N_PAD = 10240
_ROWS_TILE = _N_PAD // _NSUB
_CHUNK = 128
_CH_PER_TILE = -(-_E // (_NW * _CHUNK))
_E_TILE = _CH_PER_TILE * _CHUNK
_E_PAD = _NW * _E_TILE

_mesh = plsc.VectorSubcoreMesh(core_axis_name="c", subcore_axis_name="s")


_DEG_W = 16


def _deg_body(dst_hbm, ones_hbm, zeros_hbm, out_hbm, dst_v, ones_v, acc_sh):
    c = lax.axis_index("c")
    s = lax.axis_index("s")
    w = c * _NSUB + s

    pltpu.sync_copy(zeros_hbm.at[pl.ds(s * _ROWS_TILE, _ROWS_TILE)],
                    acc_sh.at[pl.ds(s * _ROWS_TILE, _ROWS_TILE)])
    pltpu.sync_copy(dst_hbm.at[w], dst_v)
    pltpu.sync_copy(ones_hbm, ones_v)
    plsc.subcore_barrier()

    def body(j, _):
        pltpu.sync_copy(ones_v, acc_sh.at[dst_v.at[j]], add=True)
        return 0

    lax.fori_loop(0, _CH_PER_TILE, body, 0)

    plsc.subcore_barrier()
    pltpu.sync_copy(acc_sh.at[pl.ds(s * _ROWS_TILE, _ROWS_TILE)],
                    out_hbm.at[c, pl.ds(s * _ROWS_TILE, _ROWS_TILE)])


@jax.jit
def _deg_call(dstp, ones_blk, zeros_deg):
    k = functools.partial(
        pl.kernel,
        mesh=_mesh,
        compiler_params=pltpu.CompilerParams(use_tc_tiling_on_sc=False),
        out_type=jax.ShapeDtypeStruct((_NSC, _N_PAD, _DEG_W), jnp.float32),
        scratch_types=[
            pltpu.VMEM((_CH_PER_TILE, _CHUNK), jnp.int32),
            pltpu.VMEM((_CHUNK, _DEG_W), jnp.float32),
            pltpu.VMEM_SHARED((_N_PAD, _DEG_W), jnp.float32),
        ],
    )(_deg_body)
    return k(dstp, ones_blk, zeros_deg)


def _hop_body(y_hbm, src_hbm, dst_hbm, zeros_hbm, out_hbm,
              src_v, dst_v, rows_v, acc_sh, sem):
    c = lax.axis_index("c")
    s = lax.axis_index("s")
    w = c * _NSUB + s

    pltpu.sync_copy(zeros_hbm.at[pl.ds(s * _ROWS_TILE, _ROWS_TILE)],
                    acc_sh.at[pl.ds(s * _ROWS_TILE, _ROWS_TILE)])
    pltpu.sync_copy(src_hbm.at[w], src_v)
    pltpu.sync_copy(dst_hbm.at[w], dst_v)
    plsc.subcore_barrier()

    def body(j, _):
        pltpu.async_copy(y_hbm.at[src_v.at[j]], rows_v, sem).wait()
        pltpu.sync_copy(rows_v, acc_sh.at[dst_v.at[j]], add=True)
        return 0

    lax.fori_loop(0, _CH_PER_TILE, body, 0)

    plsc.subcore_barrier()
    pltpu.sync_copy(acc_sh.at[pl.ds(s * _ROWS_TILE, _ROWS_TILE)],
                    out_hbm.at[c, pl.ds(s * _ROWS_TILE, _ROWS_TILE)])


@jax.jit
def _hop_call(y, srcp, dstp, zeros_pad):
    k = functools.partial(
        pl.kernel,
        mesh=_mesh,
        compiler_params=pltpu.CompilerParams(use_tc_tiling_on_sc=False),
        out_type=jax.ShapeDtypeStruct((_NSC, _N_PAD, _C), jnp.float32),
        scratch_types=[
            pltpu.VMEM((_CH_PER_TILE, _CHUNK), jnp.int32),
            pltpu.VMEM((_CH_PER_TILE, _CHUNK), jnp.int32),
            pltpu.VMEM((_CHUNK, _C), jnp.float32),
            pltpu.VMEM_SHARED((_N_PAD, _C), jnp.float32),
            pltpu.SemaphoreType.DMA,
        ],
    )(_hop_body)
    return k(y, srcp, dstp, zeros_pad)


_BLK = 256


def _prep_body(feat_ref, w_ref, degp_ref, y0_ref):
    deg = degp_ref[0, :, 0] + degp_ref[1, :, 0]
    norm = lax.rsqrt(jnp.maximum(deg, 1.0))
    acc = jnp.dot(feat_ref[...], w_ref[...],
                  preferred_element_type=jnp.float32)
    y0_ref[...] = acc * norm[:, None]


@jax.jit
def _prep_call(featp, W, degp):
    return pl.pallas_call(
        _prep_body,
        grid=(_N_PAD // _BLK,),
        in_specs=[
            pl.BlockSpec((_BLK, _D), lambda i: (i, 0)),
            pl.BlockSpec((_D, _C), lambda i: (0, 0)),
            pl.BlockSpec((_NSC, _BLK, _DEG_W), lambda i: (0, i, 0)),
        ],
        out_specs=pl.BlockSpec((_BLK, _C), lambda i: (i, 0)),
        out_shape=jax.ShapeDtypeStruct((_N_PAD, _C), jnp.float32),
    )(featp, W, degp)


def _mid_body(p_ref, degp_ref, y_ref):
    deg = jnp.maximum(degp_ref[0, :, 0] + degp_ref[1, :, 0], 1.0)
    y_ref[...] = (p_ref[0] + p_ref[1]) * (1.0 / deg)[:, None]


@jax.jit
def _mid_call(p, degp):
    return pl.pallas_call(
        _mid_body,
        grid=(_N_PAD // _BLK,),
        in_specs=[
            pl.BlockSpec((_NSC, _BLK, _C), lambda i: (0, i, 0)),
            pl.BlockSpec((_NSC, _BLK, _DEG_W), lambda i: (0, i, 0)),
        ],
        out_specs=pl.BlockSpec((_BLK, _C), lambda i: (i, 0)),
        out_shape=jax.ShapeDtypeStruct((_N_PAD, _C), jnp.float32),
    )(p, degp)


def _fin_body(p_ref, degp_ref, out_ref, logits_ref):
    deg = jnp.maximum(degp_ref[0, :, 0] + degp_ref[1, :, 0], 1.0)
    norm = lax.rsqrt(deg)
    logits = (p_ref[0] + p_ref[1]) * norm[:, None]
    logits_ref[...] = logits
    m = jnp.max(logits, axis=1, keepdims=True)
    e = jnp.exp(logits - m)
    out_ref[...] = e / jnp.sum(e, axis=1, keepdims=True)


@jax.jit
def _fin_call(p, degp):
    return pl.pallas_call(
        _fin_body,
        grid=(_N_PAD // _BLK,),
        in_specs=[
            pl.BlockSpec((_NSC, _BLK, _C), lambda i: (0, i, 0)),
            pl.BlockSpec((_NSC, _BLK, _DEG_W), lambda i: (0, i, 0)),
        ],
        out_specs=[
            pl.BlockSpec((_BLK, _C), lambda i: (i, 0)),
            pl.BlockSpec((_BLK, _C), lambda i: (i, 0)),
        ],
        out_shape=[
            jax.ShapeDtypeStruct((_N_PAD, _C), jnp.float32),
            jax.ShapeDtypeStruct((_N_PAD, _C), jnp.float32),
        ],
    )(p, degp)


def kernel(features, edge_index, W):
    src = edge_index[0]
    dst = edge_index[1]
    pad_idx = jnp.full((_E_PAD - _E,), _N_PAD - 1, jnp.int32)
    srcp = jnp.concatenate([src, pad_idx]).reshape(_NW, _CH_PER_TILE, _CHUNK)
    dstp = jnp.concatenate([dst, pad_idx]).reshape(_NW, _CH_PER_TILE, _CHUNK)
    featp = jnp.pad(features, ((0, _N_PAD - _N), (0, 0)))
    zeros_pad = jnp.zeros((_N_PAD, _C), jnp.float32)

    ones_blk = jnp.ones((_CHUNK, _DEG_W), jnp.float32)
    zeros_deg = jnp.zeros((_N_PAD, _DEG_W), jnp.float32)
    degp = _deg_call(dstp, ones_blk, zeros_deg)
    y0 = _prep_call(featp, W, degp)
    p1 = _hop_call(y0, srcp, dstp, zeros_pad)
    y1 = _mid_call(p1, degp)
    p2 = _hop_call(y1, srcp, dstp, zeros_pad)
    out_pad, logits_pad = _fin_call(p2, degp)
    return out_pad[:_N], logits_pad[:_N]
